# baseline (speedup 1.0000x reference)
import functools
import math

import jax
import jax.numpy as jnp
from jax import lax
from jax.experimental import pallas as pl
from jax.experimental.pallas import tpu as pltpu
from jax.experimental.pallas import tpu_sc as plsc

HID = 128
HEADS = 8
C = 16
NC = 2
NS = 16
NW = NC * NS
KA = 40
KC = 80
NEG = -3.0e38


def _ln(h, g, b):
    m = jnp.mean(h, axis=-1, keepdims=True)
    v = jnp.mean((h - m) * (h - m), axis=-1, keepdims=True)
    return (h - m) / jnp.sqrt(v + 1e-5) * g + b


def _tc_edge_body(es, w1, b1, w2, b2, g, b, we, o0, o1, o2):
    sf = jnp.dot(es[:], w1[:], preferred_element_type=jnp.float32) + b1[:]
    sf = 0.5 * sf * (1.0 + lax.erf(sf / math.sqrt(2.0)))
    sf = jnp.dot(sf, w2[:], preferred_element_type=jnp.float32) + b2[:]
    sf = _ln(sf, g[:], b[:])
    ea = jnp.dot(sf, we[:], preferred_element_type=jnp.float32)
    o0[:] = ea[:, 0:128]
    o1[:] = ea[:, 128:256]
    o2[:] = ea[:, 256:384]


def _tc_edge(edge_sig, w1, b1, w2, b2, g, b, we):
    e = edge_sig.shape[0]
    be = 512
    full = lambda shp: pl.BlockSpec(shp, lambda i: (0, 0))
    return pl.pallas_call(
        _tc_edge_body,
        grid=(e // be,),
        in_specs=[
            pl.BlockSpec((be, 16), lambda i: (i, 0)),
            full((16, 64)), full((1, 64)), full((64, 128)), full((1, 128)),
            full((1, 128)), full((1, 128)), full((128, 384)),
        ],
        out_specs=[pl.BlockSpec((be, 128), lambda i: (i, 0))] * 3,
        out_shape=[jax.ShapeDtypeStruct((e, 128), jnp.float32)] * 3,
    )(edge_sig, w1, b1, w2, b2, g, b, we)


def _tc_node0_body(x, win, bin_, g, b, wl, bl, wr, br, ho, xlo, xro):
    h = jnp.dot(x[:], win[:], preferred_element_type=jnp.float32) + bin_[:]
    hn = _ln(h, g[:], b[:])
    ho[:] = h
    xlo[:] = jnp.dot(hn, wl[:], preferred_element_type=jnp.float32) + bl[:]
    xro[:] = jnp.dot(hn, wr[:], preferred_element_type=jnp.float32) + br[:]


def _tc_node0(x, win, bin_, g, b, wl, bl, wr, br):
    n = x.shape[0]
    bn = 400
    full = lambda shp: pl.BlockSpec(shp, lambda i: (0, 0))
    return pl.pallas_call(
        _tc_node0_body,
        grid=(n // bn,),
        in_specs=[
            pl.BlockSpec((bn, 128), lambda i: (i, 0)),
            full((128, 128)), full((1, 128)), full((1, 128)), full((1, 128)),
            full((128, 128)), full((1, 128)), full((128, 128)), full((1, 128)),
        ],
        out_specs=[pl.BlockSpec((bn, 128), lambda i: (i, 0))] * 3,
        out_shape=[jax.ShapeDtypeStruct((n, 128), jnp.float32)] * 3,
    )(x, win, bin_, g, b, wl, bl, wr, br)


def _attn_from_parts(nump, denp, bias):
    num = nump[0] + nump[1]
    den = denp[0] + denp[1]
    cols = []
    for h in range(HEADS):
        cols.append(num[:, h * C:(h + 1) * C] / (den[:, h:h + 1] + 1e-16))
    return jnp.concatenate(cols, axis=1) + bias


def _tc_node_body(hp, nump, denp, bias, g, b, wl, bl, wr, br, ho, xlo, xro):
    h = hp[:] + _attn_from_parts(nump[:], denp[:], bias[:])
    hn = _ln(h, g[:], b[:])
    ho[:] = h
    xlo[:] = jnp.dot(hn, wl[:], preferred_element_type=jnp.float32) + bl[:]
    xro[:] = jnp.dot(hn, wr[:], preferred_element_type=jnp.float32) + br[:]


def _tc_node(hp, nump, denp, bias, g, b, wl, bl, wr, br):
    n = hp.shape[0]
    bn = 400
    full = lambda shp: pl.BlockSpec(shp, lambda i: (0, 0))
    return pl.pallas_call(
        _tc_node_body,
        grid=(n // bn,),
        in_specs=[
            pl.BlockSpec((bn, 128), lambda i: (i, 0)),
            pl.BlockSpec((2, bn, 128), lambda i: (0, i, 0)),
            pl.BlockSpec((2, bn, 8), lambda i: (0, i, 0)),
            full((1, 128)), full((1, 128)), full((1, 128)),
            full((128, 128)), full((1, 128)), full((128, 128)), full((1, 128)),
        ],
        out_specs=[pl.BlockSpec((bn, 128), lambda i: (i, 0))] * 3,
        out_shape=[jax.ShapeDtypeStruct((n, 128), jnp.float32)] * 3,
    )(hp, nump, denp, bias, g, b, wl, bl, wr, br)


def _tc_final_body(hp, nump, denp, bias, g, b, out):
    h = hp[:] + _attn_from_parts(nump[:], denp[:], bias[:])
    out[:] = _ln(h, g[:], b[:])


def _tc_final(hp, nump, denp, bias, g, b):
    n = hp.shape[0]
    bn = 400
    full = lambda shp: pl.BlockSpec(shp, lambda i: (0, 0))
    return pl.pallas_call(
        _tc_final_body,
        grid=(n // bn,),
        in_specs=[
            pl.BlockSpec((bn, 128), lambda i: (i, 0)),
            pl.BlockSpec((2, bn, 128), lambda i: (0, i, 0)),
            pl.BlockSpec((2, bn, 8), lambda i: (0, i, 0)),
            full((1, 128)), full((1, 128)), full((1, 128)),
        ],
        out_specs=pl.BlockSpec((bn, 128), lambda i: (i, 0)),
        out_shape=jax.ShapeDtypeStruct((n, 128), jnp.float32),
    )(hp, nump, denp, bias, g, b)


def _tc_amax_body(lmax, out):
    mx = jnp.max(lmax[:], axis=0)
    out[:] = jnp.concatenate([mx] * 16, axis=1)


def _tc_amax(lmax):
    n = lmax.shape[1]
    bn = 400
    return pl.pallas_call(
        _tc_amax_body,
        grid=(n // bn,),
        in_specs=[pl.BlockSpec((NW, bn, 8), lambda i: (0, i, 0))],
        out_specs=pl.BlockSpec((bn, 128), lambda i: (i, 0)),
        out_shape=jax.ShapeDtypeStruct((n, 128), jnp.float32),
    )(lmax)


def _sc_logits(src, dst, xl, xr, ea, att):
    e = src.shape[0]
    n = xl.shape[0]
    epw = e // NW
    nch = epw // KA
    mesh = plsc.VectorSubcoreMesh(core_axis_name="c", subcore_axis_name="s")

    buf = lambda: (pltpu.VMEM((KA,), jnp.int32), pltpu.VMEM((KA,), jnp.int32),
                   pltpu.VMEM((KA, 128), jnp.float32),
                   pltpu.VMEM((KA, 128), jnp.float32),
                   pltpu.VMEM((KA, 128), jnp.float32),
                   pltpu.VMEM((KA * 8,), jnp.float32))

    @functools.partial(
        pl.kernel,
        mesh=mesh,
        compiler_params=pltpu.CompilerParams(needs_layout_passes=False),
        out_type=(
            jax.ShapeDtypeStruct((e * 8,), jnp.float32),
            jax.ShapeDtypeStruct((NW, n * 8), jnp.float32),
        ),
        scratch_types=[
            buf(), buf(),
            pltpu.VMEM((KA * 128,), jnp.float32),
            pltpu.VMEM((128,), jnp.float32),
            pltpu.VMEM((n * 8,), jnp.float32),
            pltpu.SemaphoreType.DMA,
            pltpu.SemaphoreType.DMA,
            pltpu.SemaphoreType.DMA,
            pltpu.SemaphoreType.DMA,
            pltpu.SemaphoreType.DMA,
            pltpu.SemaphoreType.DMA,
        ],
    )
    def k(src_h, dst_h, xl_h, xr_h, ea_h, att_h, alpha_h, lmax_h,
          set0, set1, tflat, attb, amaxb, si0, si1, sg0, sg1, sa0, sa1):
        wid = lax.axis_index("s") * NC + lax.axis_index("c")
        lane = jnp.arange(16, dtype=jnp.int32)
        hsel = jnp.bitwise_and(lane, 7)
        esel = lax.shift_right_logical(lane, 3)
        e0 = wid * epw

        pltpu.sync_copy(att_h, attb)

        def init_body(i, _):
            amaxb[pl.ds(i * 16, 16)] = jnp.full((16,), NEG, jnp.float32)
            return 0
        lax.fori_loop(0, n * 8 // 16, init_body, 0)

        def issue_idx(j, st, sem):
            base = e0 + j * KA
            pltpu.async_copy(src_h.at[pl.ds(base, KA)], st[0], sem)
            pltpu.async_copy(dst_h.at[pl.ds(base, KA)], st[1], sem)

        def issue_gathers(j, st, sem):
            ids_v, idd_v, xlb, xrb, eab = st[:5]
            base = e0 + j * KA
            pltpu.async_copy(xl_h.at[ids_v], xlb, sem)
            pltpu.async_copy(xr_h.at[idd_v], xrb, sem)
            pltpu.async_copy(ea_h.at[pl.ds(base, KA)], eab, sem)

        def drain_idx(st, sem):
            pltpu.make_async_copy(src_h.at[pl.ds(0, KA)], st[0], sem).wait()
            pltpu.make_async_copy(dst_h.at[pl.ds(0, KA)], st[1], sem).wait()

        def drain_gathers(st, sem):
            pltpu.make_async_copy(xl_h.at[st[0]], st[2], sem).wait()
            pltpu.make_async_copy(xr_h.at[st[1]], st[3], sem).wait()
            pltpu.make_async_copy(ea_h.at[pl.ds(0, KA)], st[4], sem).wait()

        def compute(j, st, sa):
            _, idd_v, xlb, xrb, eab, alpb = st
            base = e0 + j * KA

            def edge_body(kk, _):
                for h in range(HEADS):
                    sl = pl.ds(h * 16, 16)
                    v = xlb[kk, sl] + xrb[kk, sl] + eab[kk, sl]
                    v = jnp.maximum(v, 0.0) + 0.2 * jnp.minimum(v, 0.0)
                    tflat[pl.ds(kk * 128 + h * 16, 16)] = v * attb[sl]
                return 0
            lax.fori_loop(0, KA, edge_body, 0, unroll=2)

            def pair_body(p, _):
                fbase = (p * 2 + esel) * 128 + hsel * 16
                acc0 = plsc.load_gather(tflat, [fbase])
                acc1 = plsc.load_gather(tflat, [fbase + 1])
                for cc in range(2, 16, 2):
                    acc0 = acc0 + plsc.load_gather(tflat, [fbase + cc])
                    acc1 = acc1 + plsc.load_gather(tflat, [fbase + cc + 1])
                acc = acc0 + acc1
                alpb[pl.ds(p * 16, 16)] = acc
                dpair = plsc.load_gather(idd_v, [p * 2 + esel])
                midx = dpair * 8 + hsel
                old = plsc.load_gather(amaxb, [midx])
                plsc.store_scatter(amaxb, [midx], jnp.maximum(old, acc))
                return 0
            lax.fori_loop(0, KA // 2, pair_body, 0, unroll=2)

            pltpu.async_copy(alpb, alpha_h.at[pl.ds(base * 8, KA * 8)], sa)

        def drain_alpha(st, sa):
            pltpu.make_async_copy(st[5], alpha_h.at[pl.ds(0, KA * 8)],
                                  sa).wait()

        issue_idx(0, set0, si0)
        drain_idx(set0, si0)
        issue_gathers(0, set0, sg0)
        issue_idx(1, set1, si1)

        def body(j2, _):
            a = j2 * 2

            @pl.when(a + 1 < nch)
            def _():
                drain_idx(set1, si1)
                issue_gathers(a + 1, set1, sg1)
            drain_gathers(set0, sg0)

            @pl.when(a >= 2)
            def _():
                drain_alpha(set0, sa0)
            compute(a, set0, sa0)

            @pl.when(a + 2 < nch)
            def _():
                issue_idx(a + 2, set0, si0)
                drain_idx(set0, si0)
                issue_gathers(a + 2, set0, sg0)

            @pl.when(a + 1 < nch)
            def _():
                drain_gathers(set1, sg1)

                @pl.when(a >= 2)
                def _():
                    drain_alpha(set1, sa1)
                compute(a + 1, set1, sa1)

            @pl.when(a + 3 < nch)
            def _():
                issue_idx(a + 3, set1, si1)
            return 0
        lax.fori_loop(0, (nch + 1) // 2, body, 0)
        drain_alpha(set0, sa0)

        @pl.when(nch > 1)
        def _():
            drain_alpha(set1, sa1)
        pltpu.sync_copy(amaxb, lmax_h.at[wid])

    return k(src, dst, xl, xr, ea, att)


def _sc_accum(src, dst, xl, alpha, amax):
    e = src.shape[0]
    n = xl.shape[0]
    epw = e // NW
    nch = epw // KC
    npsc = (n // NS) // 8 * 8
    ntail = n - npsc * NS
    nfull = npsc // KC
    nrem = npsc - nfull * KC
    nd = n * 8
    dpsc = (nd // NS) // 128 * 128
    dtail = nd - dpsc * NS
    ndq = KC * 8 // 128
    mesh = plsc.VectorSubcoreMesh(core_axis_name="c", subcore_axis_name="s")

    buf = lambda: (pltpu.VMEM((KC,), jnp.int32), pltpu.VMEM((KC,), jnp.int32),
                   pltpu.VMEM((KC, 128), jnp.float32),
                   pltpu.VMEM((KC * 8,), jnp.float32),
                   pltpu.VMEM((KC, 128), jnp.float32))

    @functools.partial(
        pl.kernel,
        mesh=mesh,
        compiler_params=pltpu.CompilerParams(needs_layout_passes=False),
        out_type=(
            jax.ShapeDtypeStruct((NC, n, 128), jnp.float32),
            jax.ShapeDtypeStruct((NC, nd), jnp.float32),
        ),
        scratch_types=[
            buf(), buf(),
            pltpu.VMEM((KC * 8 + 16,), jnp.float32),
            pltpu.VMEM((ndq, 128), jnp.int32),
            pltpu.VMEM((624,), jnp.float32),
            pltpu.VMEM_SHARED((n, 128), jnp.float32),
            pltpu.VMEM_SHARED((nd,), jnp.float32),
            pltpu.SemaphoreType.DMA,
            pltpu.SemaphoreType.DMA,
            pltpu.SemaphoreType.DMA,
            pltpu.SemaphoreType.DMA,
        ],
    )
    def k(src_h, dst_h, xl_h, alpha_h, amax_h, num_h, den_h,
          set0, set1, exb, idxd, zbuf, spnum, spden,
          si0, si1, sg0, sg1):
        cid = lax.axis_index("c")
        sid = lax.axis_index("s")
        wid = sid * NC + cid
        lane = jnp.arange(16, dtype=jnp.int32)
        hsel = jnp.bitwise_and(lane, 7)
        esel = lax.shift_right_logical(lane, 3)
        e0 = wid * epw

        zrows = set0[2]
        def zb(i, _):
            for t in range(8):
                zrows[i, pl.ds(t * 16, 16)] = jnp.zeros((16,), jnp.float32)
            return 0
        lax.fori_loop(0, KC, zb, 0)
        exb[pl.ds(KC * 8, 16)] = jnp.zeros((16,), jnp.float32)
        def zx(i, _):
            zbuf[pl.ds(i * 16, 16)] = jnp.zeros((16,), jnp.float32)
            return 0
        lax.fori_loop(0, 624 // 16, zx, 0)
        r0 = sid * npsc
        for q in range(nfull):
            pltpu.sync_copy(zrows, spnum.at[pl.ds(r0 + q * KC, KC)])
        if nrem:
            pltpu.sync_copy(zrows.at[pl.ds(0, nrem)],
                            spnum.at[pl.ds(r0 + nfull * KC, nrem)])
        if ntail:
            @pl.when(sid == NS - 1)
            def _():
                pltpu.sync_copy(zrows.at[pl.ds(0, ntail)],
                                spnum.at[pl.ds(npsc * NS, ntail)])
        d0 = sid * dpsc
        for q in range(dpsc // 624):
            pltpu.sync_copy(zbuf, spden.at[pl.ds(d0 + q * 624, 624)])
        if dtail:
            @pl.when(sid == NS - 1)
            def _():
                pltpu.sync_copy(zbuf.at[pl.ds(0, dtail)],
                                spden.at[pl.ds(dpsc * NS, dtail)])
        plsc.subcore_barrier()

        def issue_idx(j, st, sem):
            base = e0 + j * KC
            pltpu.async_copy(src_h.at[pl.ds(base, KC)], st[0], sem)
            pltpu.async_copy(dst_h.at[pl.ds(base, KC)], st[1], sem)

        def issue_gathers(j, st, sem):
            ids_v, idd_v, xlb, alpb, amr = st
            base = e0 + j * KC
            pltpu.async_copy(xl_h.at[ids_v], xlb, sem)
            pltpu.async_copy(amax_h.at[idd_v], amr, sem)
            pltpu.async_copy(alpha_h.at[pl.ds(base * 8, KC * 8)], alpb, sem)

        def drain_idx(st, sem):
            pltpu.make_async_copy(src_h.at[pl.ds(0, KC)], st[0], sem).wait()
            pltpu.make_async_copy(dst_h.at[pl.ds(0, KC)], st[1], sem).wait()

        def drain_gathers(st, sem):
            pltpu.make_async_copy(xl_h.at[st[0]], st[2], sem).wait()
            pltpu.make_async_copy(amax_h.at[st[1]], st[4], sem).wait()
            pltpu.make_async_copy(alpha_h.at[pl.ds(0, KC * 8)], st[3],
                                  sem).wait()

        def compute(j, st):
            _, idd_v, xlb, alpb, amr = st

            def pair_body(p, _):
                a = alpb[pl.ds(p * 16, 16)]
                v0 = amr[p * 2, pl.ds(0, 16)]
                v1 = amr[p * 2 + 1, pl.ds(0, 16)]
                am = jnp.where(lane < 8, v0, v1)
                ex = jnp.exp(a - am)
                exb[pl.ds(p * 16, 16)] = ex
                dpair = plsc.load_gather(idd_v, [p * 2 + esel])
                midx = dpair * 8 + hsel
                q = lax.shift_right_logical(p, 3)
                off = jnp.bitwise_and(p, 7) * 16
                idxd[q, pl.ds(off, 16)] = midx
                return 0
            lax.fori_loop(0, KC // 2, pair_body, 0, unroll=2)

            def edge_body(kk, _):
                eoff = kk * 8
                for h in range(HEADS):
                    sl = pl.ds(h * 16, 16)
                    m = plsc.load_gather(exb, [lane * 0 + (eoff + h)])
                    xlb[kk, sl] = xlb[kk, sl] * m
                return 0
            lax.fori_loop(0, KC, edge_body, 0, unroll=2)

            pltpu.sync_copy(xlb, spnum.at[idd_v], add=True)
            for q in range(ndq):
                pltpu.sync_copy(exb.at[pl.ds(q * 128, 128)],
                                spden.at[idxd.at[q]], add=True)

        issue_idx(0, set0, si0)
        drain_idx(set0, si0)
        issue_gathers(0, set0, sg0)
        issue_idx(1, set1, si1)

        def body(j2, _):
            a = j2 * 2

            @pl.when(a + 1 < nch)
            def _():
                drain_idx(set1, si1)
                issue_gathers(a + 1, set1, sg1)
            drain_gathers(set0, sg0)
            compute(a, set0)

            @pl.when(a + 2 < nch)
            def _():
                issue_idx(a + 2, set0, si0)
                drain_idx(set0, si0)
                issue_gathers(a + 2, set0, sg0)

            @pl.when(a + 1 < nch)
            def _():
                drain_gathers(set1, sg1)
                compute(a + 1, set1)

            @pl.when(a + 3 < nch)
            def _():
                issue_idx(a + 3, set1, si1)
            return 0
        lax.fori_loop(0, (nch + 1) // 2, body, 0)
        plsc.subcore_barrier()

        pltpu.sync_copy(spnum.at[pl.ds(r0, npsc)],
                        num_h.at[cid, pl.ds(r0, npsc)])
        if ntail:
            @pl.when(sid == NS - 1)
            def _():
                pltpu.sync_copy(spnum.at[pl.ds(npsc * NS, ntail)],
                                num_h.at[cid, pl.ds(npsc * NS, ntail)])
        pltpu.sync_copy(spden.at[pl.ds(d0, dpsc)],
                        den_h.at[cid, pl.ds(d0, dpsc)])
        if dtail:
            @pl.when(sid == NS - 1)
            def _():
                pltpu.sync_copy(spden.at[pl.ds(dpsc * NS, dtail)],
                                den_h.at[cid, pl.ds(dpsc * NS, dtail)])

    return k(src, dst, xl, alpha, amax)


def kernel(x, edge_index, edge_sig, params):
    p = params
    r2 = lambda a: a.reshape(1, -1)
    src = edge_index[0]
    dst = edge_index[1]

    we_all = jnp.concatenate([lp['W_e'] for lp in p['layers']], axis=1)
    eas = _tc_edge(edge_sig, p['W_s1'], r2(p['b_s1']), p['W_s2'],
                   r2(p['b_s2']), r2(p['ln_s_g']), r2(p['ln_s_b']), we_all)

    l0 = p['layers'][0]
    h, xl, xr = _tc_node0(x, p['W_in'], r2(p['b_in']), r2(l0['ln_g']),
                          r2(l0['ln_b']), l0['W_l'], r2(l0['b_l']),
                          l0['W_r'], r2(l0['b_r']))

    n = x.shape[0]
    for l, lp in enumerate(p['layers']):
        att = lp['att'].reshape(-1)
        alpha, lmax = _sc_logits(src, dst, xl, xr, eas[l], att)
        amax = _tc_amax(lmax.reshape(NW, n, 8))
        num, den = _sc_accum(src, dst, xl, alpha, amax)
        den = den.reshape(NC, n, 8)
        if l + 1 < len(p['layers']):
            np_ = p['layers'][l + 1]
            h, xl, xr = _tc_node(h, num, den, r2(lp['bias']), r2(np_['ln_g']),
                                 r2(np_['ln_b']), np_['W_l'], r2(np_['b_l']),
                                 np_['W_r'], r2(np_['b_r']))
        else:
            out = _tc_final(h, num, den, r2(lp['bias']), r2(p['ln_f_g']),
                            r2(p['ln_f_b']))
    return out


# speedup vs baseline: 1.2984x; 1.2984x over previous
import functools
import math

import jax
import jax.numpy as jnp
from jax import lax
from jax.experimental import pallas as pl
from jax.experimental.pallas import tpu as pltpu
from jax.experimental.pallas import tpu_sc as plsc

HID = 128
HEADS = 8
C = 16
NC = 2
NS = 16
NW = NC * NS
KA = 40
KC = 80
NEG = -3.0e38


def _ln(h, g, b):
    m = jnp.mean(h, axis=-1, keepdims=True)
    v = jnp.mean((h - m) * (h - m), axis=-1, keepdims=True)
    return (h - m) / jnp.sqrt(v + 1e-5) * g + b


def _tc_edge_body(es, w1, b1, w2, b2, g, b, we, o0, o1, o2):
    sf = jnp.dot(es[:], w1[:], preferred_element_type=jnp.float32) + b1[:]
    sf = 0.5 * sf * (1.0 + lax.erf(sf / math.sqrt(2.0)))
    sf = jnp.dot(sf, w2[:], preferred_element_type=jnp.float32) + b2[:]
    sf = _ln(sf, g[:], b[:])
    ea = jnp.dot(sf, we[:], preferred_element_type=jnp.float32)
    o0[:] = ea[:, 0:128]
    o1[:] = ea[:, 128:256]
    o2[:] = ea[:, 256:384]


def _tc_edge(edge_sig, w1, b1, w2, b2, g, b, we):
    e = edge_sig.shape[0]
    be = 1280
    full = lambda shp: pl.BlockSpec(shp, lambda i: (0, 0))
    return pl.pallas_call(
        _tc_edge_body,
        grid=(e // be,),
        in_specs=[
            pl.BlockSpec((be, 16), lambda i: (i, 0)),
            full((16, 64)), full((1, 64)), full((64, 128)), full((1, 128)),
            full((1, 128)), full((1, 128)), full((128, 384)),
        ],
        out_specs=[pl.BlockSpec((be, 128), lambda i: (i, 0))] * 3,
        out_shape=[jax.ShapeDtypeStruct((e, 128), jnp.float32)] * 3,
    )(edge_sig, w1, b1, w2, b2, g, b, we)


def _tc_node0_body(x, win, bin_, g, b, wl, bl, wr, br, ho, xlo, xro):
    h = jnp.dot(x[:], win[:], preferred_element_type=jnp.float32) + bin_[:]
    hn = _ln(h, g[:], b[:])
    ho[:] = h
    xlo[:] = jnp.dot(hn, wl[:], preferred_element_type=jnp.float32) + bl[:]
    xro[:] = jnp.dot(hn, wr[:], preferred_element_type=jnp.float32) + br[:]


def _tc_node0(x, win, bin_, g, b, wl, bl, wr, br):
    n = x.shape[0]
    bn = 1000
    full = lambda shp: pl.BlockSpec(shp, lambda i: (0, 0))
    return pl.pallas_call(
        _tc_node0_body,
        grid=(n // bn,),
        in_specs=[
            pl.BlockSpec((bn, 128), lambda i: (i, 0)),
            full((128, 128)), full((1, 128)), full((1, 128)), full((1, 128)),
            full((128, 128)), full((1, 128)), full((128, 128)), full((1, 128)),
        ],
        out_specs=[pl.BlockSpec((bn, 128), lambda i: (i, 0))] * 3,
        out_shape=[jax.ShapeDtypeStruct((n, 128), jnp.float32)] * 3,
    )(x, win, bin_, g, b, wl, bl, wr, br)


def _attn_from_parts(nump, denp, bias):
    num = nump[0] + nump[1]
    den = denp[0] + denp[1]
    cols = []
    for h in range(HEADS):
        cols.append(num[:, h * C:(h + 1) * C] / (den[:, h:h + 1] + 1e-16))
    return jnp.concatenate(cols, axis=1) + bias


def _tc_node_body(hp, nump, denp, bias, g, b, wl, bl, wr, br, ho, xlo, xro):
    h = hp[:] + _attn_from_parts(nump[:], denp[:], bias[:])
    hn = _ln(h, g[:], b[:])
    ho[:] = h
    xlo[:] = jnp.dot(hn, wl[:], preferred_element_type=jnp.float32) + bl[:]
    xro[:] = jnp.dot(hn, wr[:], preferred_element_type=jnp.float32) + br[:]


def _tc_node(hp, nump, denp, bias, g, b, wl, bl, wr, br):
    n = hp.shape[0]
    bn = 1000
    full = lambda shp: pl.BlockSpec(shp, lambda i: (0, 0))
    return pl.pallas_call(
        _tc_node_body,
        grid=(n // bn,),
        in_specs=[
            pl.BlockSpec((bn, 128), lambda i: (i, 0)),
            pl.BlockSpec((2, bn, 128), lambda i: (0, i, 0)),
            pl.BlockSpec((2, bn, 8), lambda i: (0, i, 0)),
            full((1, 128)), full((1, 128)), full((1, 128)),
            full((128, 128)), full((1, 128)), full((128, 128)), full((1, 128)),
        ],
        out_specs=[pl.BlockSpec((bn, 128), lambda i: (i, 0))] * 3,
        out_shape=[jax.ShapeDtypeStruct((n, 128), jnp.float32)] * 3,
    )(hp, nump, denp, bias, g, b, wl, bl, wr, br)


def _tc_final_body(hp, nump, denp, bias, g, b, out):
    h = hp[:] + _attn_from_parts(nump[:], denp[:], bias[:])
    out[:] = _ln(h, g[:], b[:])


def _tc_final(hp, nump, denp, bias, g, b):
    n = hp.shape[0]
    bn = 1000
    full = lambda shp: pl.BlockSpec(shp, lambda i: (0, 0))
    return pl.pallas_call(
        _tc_final_body,
        grid=(n // bn,),
        in_specs=[
            pl.BlockSpec((bn, 128), lambda i: (i, 0)),
            pl.BlockSpec((2, bn, 128), lambda i: (0, i, 0)),
            pl.BlockSpec((2, bn, 8), lambda i: (0, i, 0)),
            full((1, 128)), full((1, 128)), full((1, 128)),
        ],
        out_specs=pl.BlockSpec((bn, 128), lambda i: (i, 0)),
        out_shape=jax.ShapeDtypeStruct((n, 128), jnp.float32),
    )(hp, nump, denp, bias, g, b)


def _tc_amax_body(lmax, out):
    mx = jnp.max(lmax[:], axis=0)
    out[:] = jnp.concatenate([mx] * 16, axis=1)


def _tc_amax(lmax):
    n = lmax.shape[1]
    bn = 1000
    return pl.pallas_call(
        _tc_amax_body,
        grid=(n // bn,),
        in_specs=[pl.BlockSpec((NW, bn, 8), lambda i: (0, i, 0))],
        out_specs=pl.BlockSpec((bn, 128), lambda i: (i, 0)),
        out_shape=jax.ShapeDtypeStruct((n, 128), jnp.float32),
    )(lmax)


def _sc_logits(src, dst, xl, xr, ea, att):
    e = src.shape[0]
    n = xl.shape[0]
    epw = e // NW
    nch = epw // KA
    mesh = plsc.VectorSubcoreMesh(core_axis_name="c", subcore_axis_name="s")

    buf = lambda: (pltpu.VMEM((KA,), jnp.int32), pltpu.VMEM((KA,), jnp.int32),
                   pltpu.VMEM((KA, 128), jnp.float32),
                   pltpu.VMEM((KA, 128), jnp.float32),
                   pltpu.VMEM((KA, 128), jnp.float32),
                   pltpu.VMEM((KA * 8,), jnp.float32))

    @functools.partial(
        pl.kernel,
        mesh=mesh,
        compiler_params=pltpu.CompilerParams(needs_layout_passes=False),
        out_type=(
            jax.ShapeDtypeStruct((e * 8,), jnp.float32),
            jax.ShapeDtypeStruct((NW, n * 8), jnp.float32),
        ),
        scratch_types=[
            buf(), buf(),
            pltpu.VMEM((KA * 128,), jnp.float32),
            pltpu.VMEM((128,), jnp.float32),
            pltpu.VMEM((n * 8,), jnp.float32),
            pltpu.SemaphoreType.DMA,
            pltpu.SemaphoreType.DMA,
            pltpu.SemaphoreType.DMA,
            pltpu.SemaphoreType.DMA,
            pltpu.SemaphoreType.DMA,
            pltpu.SemaphoreType.DMA,
        ],
    )
    def k(src_h, dst_h, xl_h, xr_h, ea_h, att_h, alpha_h, lmax_h,
          set0, set1, tflat, attb, amaxb, si0, si1, sg0, sg1, sa0, sa1):
        wid = lax.axis_index("s") * NC + lax.axis_index("c")
        lane = jnp.arange(16, dtype=jnp.int32)
        hsel = jnp.bitwise_and(lane, 7)
        esel = lax.shift_right_logical(lane, 3)
        e0 = wid * epw

        pltpu.sync_copy(att_h, attb)

        def init_body(i, _):
            amaxb[pl.ds(i * 16, 16)] = jnp.full((16,), NEG, jnp.float32)
            return 0
        lax.fori_loop(0, n * 8 // 16, init_body, 0)

        def issue_idx(j, st, sem):
            base = e0 + j * KA
            pltpu.async_copy(src_h.at[pl.ds(base, KA)], st[0], sem)
            pltpu.async_copy(dst_h.at[pl.ds(base, KA)], st[1], sem)

        def issue_gathers(j, st, sem):
            ids_v, idd_v, xlb, xrb, eab = st[:5]
            base = e0 + j * KA
            pltpu.async_copy(xl_h.at[ids_v], xlb, sem)
            pltpu.async_copy(xr_h.at[idd_v], xrb, sem)
            pltpu.async_copy(ea_h.at[pl.ds(base, KA)], eab, sem)

        def drain_idx(st, sem):
            pltpu.make_async_copy(src_h.at[pl.ds(0, KA)], st[0], sem).wait()
            pltpu.make_async_copy(dst_h.at[pl.ds(0, KA)], st[1], sem).wait()

        def drain_gathers(st, sem):
            pltpu.make_async_copy(xl_h.at[st[0]], st[2], sem).wait()
            pltpu.make_async_copy(xr_h.at[st[1]], st[3], sem).wait()
            pltpu.make_async_copy(ea_h.at[pl.ds(0, KA)], st[4], sem).wait()

        def compute(j, st, sa):
            _, idd_v, xlb, xrb, eab, alpb = st
            base = e0 + j * KA

            def edge_body(kk, _):
                for h in range(HEADS):
                    sl = pl.ds(h * 16, 16)
                    v = xlb[kk, sl] + xrb[kk, sl] + eab[kk, sl]
                    v = jnp.maximum(v, 0.0) + 0.2 * jnp.minimum(v, 0.0)
                    tflat[pl.ds(kk * 128 + h * 16, 16)] = v * attb[sl]
                return 0
            lax.fori_loop(0, KA, edge_body, 0)

            def pair_body(p, _):
                fbase = (p * 2 + esel) * 128 + hsel * 16
                acc0 = plsc.load_gather(tflat, [fbase])
                acc1 = plsc.load_gather(tflat, [fbase + 1])
                for cc in range(2, 16, 2):
                    acc0 = acc0 + plsc.load_gather(tflat, [fbase + cc])
                    acc1 = acc1 + plsc.load_gather(tflat, [fbase + cc + 1])
                acc = acc0 + acc1
                alpb[pl.ds(p * 16, 16)] = acc
                dpair = plsc.load_gather(idd_v, [p * 2 + esel])
                midx = dpair * 8 + hsel
                old = plsc.load_gather(amaxb, [midx])
                plsc.store_scatter(amaxb, [midx], jnp.maximum(old, acc))
                return 0
            lax.fori_loop(0, KA // 2, pair_body, 0)

            pltpu.async_copy(alpb, alpha_h.at[pl.ds(base * 8, KA * 8)], sa)

        def drain_alpha(st, sa):
            pltpu.make_async_copy(st[5], alpha_h.at[pl.ds(0, KA * 8)],
                                  sa).wait()

        issue_idx(0, set0, si0)
        drain_idx(set0, si0)
        issue_gathers(0, set0, sg0)
        issue_idx(1, set1, si1)

        def body(j2, _):
            a = j2 * 2

            @pl.when(a + 1 < nch)
            def _():
                drain_idx(set1, si1)
                issue_gathers(a + 1, set1, sg1)
            drain_gathers(set0, sg0)

            @pl.when(a >= 2)
            def _():
                drain_alpha(set0, sa0)
            compute(a, set0, sa0)

            @pl.when(a + 2 < nch)
            def _():
                issue_idx(a + 2, set0, si0)
                drain_idx(set0, si0)
                issue_gathers(a + 2, set0, sg0)

            @pl.when(a + 1 < nch)
            def _():
                drain_gathers(set1, sg1)

                @pl.when(a >= 2)
                def _():
                    drain_alpha(set1, sa1)
                compute(a + 1, set1, sa1)

            @pl.when(a + 3 < nch)
            def _():
                issue_idx(a + 3, set1, si1)
            return 0
        lax.fori_loop(0, (nch + 1) // 2, body, 0)
        drain_alpha(set0, sa0)

        @pl.when(nch > 1)
        def _():
            drain_alpha(set1, sa1)
        pltpu.sync_copy(amaxb, lmax_h.at[wid])

    return k(src, dst, xl, xr, ea, att)


def _sc_accum(src, dst, xl, alpha, amax):
    e = src.shape[0]
    n = xl.shape[0]
    epw = e // NW
    nch = epw // KC
    npsc = (n // NS) // 8 * 8
    ntail = n - npsc * NS
    nfull = npsc // KC
    nrem = npsc - nfull * KC
    nd = n * 8
    dpsc = (nd // NS) // 128 * 128
    dtail = nd - dpsc * NS
    ndq = KC * 8 // 128
    mesh = plsc.VectorSubcoreMesh(core_axis_name="c", subcore_axis_name="s")

    buf = lambda: (pltpu.VMEM((KC,), jnp.int32), pltpu.VMEM((KC,), jnp.int32),
                   pltpu.VMEM((KC, 128), jnp.float32),
                   pltpu.VMEM((KC * 8,), jnp.float32),
                   pltpu.VMEM((KC, 128), jnp.float32))

    @functools.partial(
        pl.kernel,
        mesh=mesh,
        compiler_params=pltpu.CompilerParams(needs_layout_passes=False),
        out_type=(
            jax.ShapeDtypeStruct((NC, n, 128), jnp.float32),
            jax.ShapeDtypeStruct((NC, nd), jnp.float32),
        ),
        scratch_types=[
            buf(), buf(),
            pltpu.VMEM((KC * 8 + 16,), jnp.float32),
            pltpu.VMEM((ndq, 128), jnp.int32),
            pltpu.VMEM((624,), jnp.float32),
            pltpu.VMEM_SHARED((n, 128), jnp.float32),
            pltpu.VMEM_SHARED((nd,), jnp.float32),
            pltpu.SemaphoreType.DMA,
            pltpu.SemaphoreType.DMA,
            pltpu.SemaphoreType.DMA,
            pltpu.SemaphoreType.DMA,
        ],
    )
    def k(src_h, dst_h, xl_h, alpha_h, amax_h, num_h, den_h,
          set0, set1, exb, idxd, zbuf, spnum, spden,
          si0, si1, sg0, sg1):
        cid = lax.axis_index("c")
        sid = lax.axis_index("s")
        wid = sid * NC + cid
        lane = jnp.arange(16, dtype=jnp.int32)
        hsel = jnp.bitwise_and(lane, 7)
        esel = lax.shift_right_logical(lane, 3)
        e0 = wid * epw

        zrows = set0[2]
        def zb(i, _):
            for t in range(8):
                zrows[i, pl.ds(t * 16, 16)] = jnp.zeros((16,), jnp.float32)
            return 0
        lax.fori_loop(0, KC, zb, 0)
        exb[pl.ds(KC * 8, 16)] = jnp.zeros((16,), jnp.float32)
        def zx(i, _):
            zbuf[pl.ds(i * 16, 16)] = jnp.zeros((16,), jnp.float32)
            return 0
        lax.fori_loop(0, 624 // 16, zx, 0)
        r0 = sid * npsc
        for q in range(nfull):
            pltpu.sync_copy(zrows, spnum.at[pl.ds(r0 + q * KC, KC)])
        if nrem:
            pltpu.sync_copy(zrows.at[pl.ds(0, nrem)],
                            spnum.at[pl.ds(r0 + nfull * KC, nrem)])
        if ntail:
            @pl.when(sid == NS - 1)
            def _():
                pltpu.sync_copy(zrows.at[pl.ds(0, ntail)],
                                spnum.at[pl.ds(npsc * NS, ntail)])
        d0 = sid * dpsc
        for q in range(dpsc // 624):
            pltpu.sync_copy(zbuf, spden.at[pl.ds(d0 + q * 624, 624)])
        if dtail:
            @pl.when(sid == NS - 1)
            def _():
                pltpu.sync_copy(zbuf.at[pl.ds(0, dtail)],
                                spden.at[pl.ds(dpsc * NS, dtail)])
        plsc.subcore_barrier()

        def issue_idx(j, st, sem):
            base = e0 + j * KC
            pltpu.async_copy(src_h.at[pl.ds(base, KC)], st[0], sem)
            pltpu.async_copy(dst_h.at[pl.ds(base, KC)], st[1], sem)

        def issue_gathers(j, st, sem):
            ids_v, idd_v, xlb, alpb, amr = st
            base = e0 + j * KC
            pltpu.async_copy(xl_h.at[ids_v], xlb, sem)
            pltpu.async_copy(amax_h.at[idd_v], amr, sem)
            pltpu.async_copy(alpha_h.at[pl.ds(base * 8, KC * 8)], alpb, sem)

        def drain_idx(st, sem):
            pltpu.make_async_copy(src_h.at[pl.ds(0, KC)], st[0], sem).wait()
            pltpu.make_async_copy(dst_h.at[pl.ds(0, KC)], st[1], sem).wait()

        def drain_gathers(st, sem):
            pltpu.make_async_copy(xl_h.at[st[0]], st[2], sem).wait()
            pltpu.make_async_copy(amax_h.at[st[1]], st[4], sem).wait()
            pltpu.make_async_copy(alpha_h.at[pl.ds(0, KC * 8)], st[3],
                                  sem).wait()

        def compute(j, st):
            _, idd_v, xlb, alpb, amr = st

            def pair_body(p, _):
                a = alpb[pl.ds(p * 16, 16)]
                v0 = amr[p * 2, pl.ds(0, 16)]
                v1 = amr[p * 2 + 1, pl.ds(0, 16)]
                am = jnp.where(lane < 8, v0, v1)
                ex = jnp.exp(a - am)
                exb[pl.ds(p * 16, 16)] = ex
                dpair = plsc.load_gather(idd_v, [p * 2 + esel])
                midx = dpair * 8 + hsel
                q = lax.shift_right_logical(p, 3)
                off = jnp.bitwise_and(p, 7) * 16
                idxd[q, pl.ds(off, 16)] = midx
                return 0
            lax.fori_loop(0, KC // 2, pair_body, 0)

            def edge_body(kk, _):
                eoff = kk * 8
                for h in range(HEADS):
                    sl = pl.ds(h * 16, 16)
                    m = plsc.load_gather(exb, [lane * 0 + (eoff + h)])
                    xlb[kk, sl] = xlb[kk, sl] * m
                return 0
            lax.fori_loop(0, KC, edge_body, 0)

            pltpu.sync_copy(xlb, spnum.at[idd_v], add=True)
            for q in range(ndq):
                pltpu.sync_copy(exb.at[pl.ds(q * 128, 128)],
                                spden.at[idxd.at[q]], add=True)

        issue_idx(0, set0, si0)
        drain_idx(set0, si0)
        issue_gathers(0, set0, sg0)
        issue_idx(1, set1, si1)

        def body(j2, _):
            a = j2 * 2

            @pl.when(a + 1 < nch)
            def _():
                drain_idx(set1, si1)
                issue_gathers(a + 1, set1, sg1)
            drain_gathers(set0, sg0)
            compute(a, set0)

            @pl.when(a + 2 < nch)
            def _():
                issue_idx(a + 2, set0, si0)
                drain_idx(set0, si0)
                issue_gathers(a + 2, set0, sg0)

            @pl.when(a + 1 < nch)
            def _():
                drain_gathers(set1, sg1)
                compute(a + 1, set1)

            @pl.when(a + 3 < nch)
            def _():
                issue_idx(a + 3, set1, si1)
            return 0
        lax.fori_loop(0, (nch + 1) // 2, body, 0)
        plsc.subcore_barrier()

        pltpu.sync_copy(spnum.at[pl.ds(r0, npsc)],
                        num_h.at[cid, pl.ds(r0, npsc)])
        if ntail:
            @pl.when(sid == NS - 1)
            def _():
                pltpu.sync_copy(spnum.at[pl.ds(npsc * NS, ntail)],
                                num_h.at[cid, pl.ds(npsc * NS, ntail)])
        pltpu.sync_copy(spden.at[pl.ds(d0, dpsc)],
                        den_h.at[cid, pl.ds(d0, dpsc)])
        if dtail:
            @pl.when(sid == NS - 1)
            def _():
                pltpu.sync_copy(spden.at[pl.ds(dpsc * NS, dtail)],
                                den_h.at[cid, pl.ds(dpsc * NS, dtail)])

    return k(src, dst, xl, alpha, amax)


def kernel(x, edge_index, edge_sig, params):
    p = params
    r2 = lambda a: a.reshape(1, -1)
    src = edge_index[0]
    dst = edge_index[1]

    we_all = jnp.concatenate([lp['W_e'] for lp in p['layers']], axis=1)
    eas = _tc_edge(edge_sig, p['W_s1'], r2(p['b_s1']), p['W_s2'],
                   r2(p['b_s2']), r2(p['ln_s_g']), r2(p['ln_s_b']), we_all)

    l0 = p['layers'][0]
    h, xl, xr = _tc_node0(x, p['W_in'], r2(p['b_in']), r2(l0['ln_g']),
                          r2(l0['ln_b']), l0['W_l'], r2(l0['b_l']),
                          l0['W_r'], r2(l0['b_r']))

    n = x.shape[0]
    for l, lp in enumerate(p['layers']):
        att = lp['att'].reshape(-1)
        alpha, lmax = _sc_logits(src, dst, xl, xr, eas[l], att)
        amax = _tc_amax(lmax.reshape(NW, n, 8))
        num, den = _sc_accum(src, dst, xl, alpha, amax)
        den = den.reshape(NC, n, 8)
        if l + 1 < len(p['layers']):
            np_ = p['layers'][l + 1]
            h, xl, xr = _tc_node(h, num, den, r2(lp['bias']), r2(np_['ln_g']),
                                 r2(np_['ln_b']), np_['W_l'], r2(np_['b_l']),
                                 np_['W_r'], r2(np_['b_r']))
        else:
            out = _tc_final(h, num, den, r2(lp['bias']), r2(p['ln_f_g']),
                            r2(p['ln_f_b']))
    return out


# speedup vs baseline: 1.3114x; 1.0100x over previous
import functools
import math

import jax
import jax.numpy as jnp
from jax import lax
from jax.experimental import pallas as pl
from jax.experimental.pallas import tpu as pltpu
from jax.experimental.pallas import tpu_sc as plsc

HID = 128
HEADS = 8
C = 16
NC = 2
NS = 16
NW = NC * NS
KA = 40
KC = 80
NEG = -3.0e38


def _ln(h, g, b):
    m = jnp.mean(h, axis=-1, keepdims=True)
    v = jnp.mean((h - m) * (h - m), axis=-1, keepdims=True)
    return (h - m) / jnp.sqrt(v + 1e-5) * g + b


def _tc_edge_body(es, w1, b1, w2, b2, g, b, we, o0, o1, o2):
    sf = jnp.dot(es[:], w1[:], preferred_element_type=jnp.float32) + b1[:]
    sf = 0.5 * sf * (1.0 + lax.erf(sf / math.sqrt(2.0)))
    sf = jnp.dot(sf, w2[:], preferred_element_type=jnp.float32) + b2[:]
    sf = _ln(sf, g[:], b[:])
    ea = jnp.dot(sf, we[:], preferred_element_type=jnp.float32)
    now = 0
    for o in (o0, o1, o2):
        if o is not None:
            o[:] = ea[:, now:now + 128]
            now += 128


def _tc_edge(edge_sig, w1, b1, w2, b2, g, b, we):
    e = edge_sig.shape[0]
    nl = we.shape[1] // 128
    be = 1280
    full = lambda shp: pl.BlockSpec(shp, lambda i: (0, 0))
    body = lambda es, w1_, b1_, w2_, b2_, g_, b_, we_, *outs: _tc_edge_body(
        es, w1_, b1_, w2_, b2_, g_, b_, we_,
        *(list(outs) + [None] * (3 - nl)))
    return pl.pallas_call(
        body,
        grid=(e // be,),
        in_specs=[
            pl.BlockSpec((be, 16), lambda i: (i, 0)),
            full((16, 64)), full((1, 64)), full((64, 128)), full((1, 128)),
            full((1, 128)), full((1, 128)), full((128, 128 * nl)),
        ],
        out_specs=[pl.BlockSpec((be, 128), lambda i: (i, 0))] * nl,
        out_shape=[jax.ShapeDtypeStruct((e, 128), jnp.float32)] * nl,
    )(edge_sig, w1, b1, w2, b2, g, b, we)


def _tc_node0_body(x, win, bin_, g, b, wl, bl, wr, br, ho, xlo, xro):
    h = jnp.dot(x[:], win[:], preferred_element_type=jnp.float32) + bin_[:]
    hn = _ln(h, g[:], b[:])
    ho[:] = h
    xlo[:] = jnp.dot(hn, wl[:], preferred_element_type=jnp.float32) + bl[:]
    xro[:] = jnp.dot(hn, wr[:], preferred_element_type=jnp.float32) + br[:]


def _tc_node0(x, win, bin_, g, b, wl, bl, wr, br):
    n = x.shape[0]
    bn = 1000
    full = lambda shp: pl.BlockSpec(shp, lambda i: (0, 0))
    return pl.pallas_call(
        _tc_node0_body,
        grid=(n // bn,),
        in_specs=[
            pl.BlockSpec((bn, 128), lambda i: (i, 0)),
            full((128, 128)), full((1, 128)), full((1, 128)), full((1, 128)),
            full((128, 128)), full((1, 128)), full((128, 128)), full((1, 128)),
        ],
        out_specs=[pl.BlockSpec((bn, 128), lambda i: (i, 0))] * 3,
        out_shape=[jax.ShapeDtypeStruct((n, 128), jnp.float32)] * 3,
    )(x, win, bin_, g, b, wl, bl, wr, br)


def _attn_from_parts(nump, denp, bias):
    num = nump[0] + nump[1]
    den = denp[0] + denp[1]
    cols = []
    for h in range(HEADS):
        cols.append(num[:, h * C:(h + 1) * C] / (den[:, h:h + 1] + 1e-16))
    return jnp.concatenate(cols, axis=1) + bias


def _tc_node_body(hp, nump, denp, bias, g, b, wl, bl, wr, br, ho, xlo, xro):
    h = hp[:] + _attn_from_parts(nump[:], denp[:], bias[:])
    hn = _ln(h, g[:], b[:])
    ho[:] = h
    xlo[:] = jnp.dot(hn, wl[:], preferred_element_type=jnp.float32) + bl[:]
    xro[:] = jnp.dot(hn, wr[:], preferred_element_type=jnp.float32) + br[:]


def _tc_node(hp, nump, denp, bias, g, b, wl, bl, wr, br):
    n = hp.shape[0]
    bn = 1000
    full = lambda shp: pl.BlockSpec(shp, lambda i: (0, 0))
    return pl.pallas_call(
        _tc_node_body,
        grid=(n // bn,),
        in_specs=[
            pl.BlockSpec((bn, 128), lambda i: (i, 0)),
            pl.BlockSpec((2, bn, 128), lambda i: (0, i, 0)),
            pl.BlockSpec((2, bn, 8), lambda i: (0, i, 0)),
            full((1, 128)), full((1, 128)), full((1, 128)),
            full((128, 128)), full((1, 128)), full((128, 128)), full((1, 128)),
        ],
        out_specs=[pl.BlockSpec((bn, 128), lambda i: (i, 0))] * 3,
        out_shape=[jax.ShapeDtypeStruct((n, 128), jnp.float32)] * 3,
    )(hp, nump, denp, bias, g, b, wl, bl, wr, br)


def _tc_final_body(hp, nump, denp, bias, g, b, out):
    h = hp[:] + _attn_from_parts(nump[:], denp[:], bias[:])
    out[:] = _ln(h, g[:], b[:])


def _tc_final(hp, nump, denp, bias, g, b):
    n = hp.shape[0]
    bn = 1000
    full = lambda shp: pl.BlockSpec(shp, lambda i: (0, 0))
    return pl.pallas_call(
        _tc_final_body,
        grid=(n // bn,),
        in_specs=[
            pl.BlockSpec((bn, 128), lambda i: (i, 0)),
            pl.BlockSpec((2, bn, 128), lambda i: (0, i, 0)),
            pl.BlockSpec((2, bn, 8), lambda i: (0, i, 0)),
            full((1, 128)), full((1, 128)), full((1, 128)),
        ],
        out_specs=pl.BlockSpec((bn, 128), lambda i: (i, 0)),
        out_shape=jax.ShapeDtypeStruct((n, 128), jnp.float32),
    )(hp, nump, denp, bias, g, b)


def _tc_amax_body(lmax, out):
    mx = jnp.max(lmax[:], axis=0)
    out[:] = jnp.concatenate([mx] * 16, axis=1)


def _tc_amax(lmax):
    n = lmax.shape[1]
    bn = 1000
    return pl.pallas_call(
        _tc_amax_body,
        grid=(n // bn,),
        in_specs=[pl.BlockSpec((NW, bn, 8), lambda i: (0, i, 0))],
        out_specs=pl.BlockSpec((bn, 128), lambda i: (i, 0)),
        out_shape=jax.ShapeDtypeStruct((n, 128), jnp.float32),
    )(lmax)


def _sc_logits(src, dst, xl, xr, ea, att):
    e = src.shape[0]
    n = xl.shape[0]
    epw = e // NW
    nch = epw // KA
    mesh = plsc.VectorSubcoreMesh(core_axis_name="c", subcore_axis_name="s")

    buf = lambda: (pltpu.VMEM((KA,), jnp.int32), pltpu.VMEM((KA,), jnp.int32),
                   pltpu.VMEM((KA, 128), jnp.float32),
                   pltpu.VMEM((KA, 128), jnp.float32),
                   pltpu.VMEM((KA, 128), jnp.float32),
                   pltpu.VMEM((KA * 8,), jnp.float32))

    @functools.partial(
        pl.kernel,
        mesh=mesh,
        compiler_params=pltpu.CompilerParams(needs_layout_passes=False),
        out_type=(
            jax.ShapeDtypeStruct((e * 8,), jnp.float32),
            jax.ShapeDtypeStruct((NW, n * 8), jnp.float32),
        ),
        scratch_types=[
            buf(), buf(),
            pltpu.VMEM((KA * 128,), jnp.float32),
            pltpu.VMEM((128,), jnp.float32),
            pltpu.VMEM((n * 8,), jnp.float32),
            pltpu.SemaphoreType.DMA,
            pltpu.SemaphoreType.DMA,
            pltpu.SemaphoreType.DMA,
            pltpu.SemaphoreType.DMA,
            pltpu.SemaphoreType.DMA,
            pltpu.SemaphoreType.DMA,
        ],
    )
    def k(src_h, dst_h, xl_h, xr_h, ea_h, att_h, alpha_h, lmax_h,
          set0, set1, tflat, attb, amaxb, si0, si1, sg0, sg1, sa0, sa1):
        wid = lax.axis_index("s") * NC + lax.axis_index("c")
        lane = jnp.arange(16, dtype=jnp.int32)
        hsel = jnp.bitwise_and(lane, 7)
        esel = lax.shift_right_logical(lane, 3)
        e0 = wid * epw

        pltpu.sync_copy(att_h, attb)

        def init_body(i, _):
            amaxb[pl.ds(i * 16, 16)] = jnp.full((16,), NEG, jnp.float32)
            return 0
        lax.fori_loop(0, n * 8 // 16, init_body, 0)

        def issue_idx(j, st, sem):
            base = e0 + j * KA
            pltpu.async_copy(src_h.at[pl.ds(base, KA)], st[0], sem)
            pltpu.async_copy(dst_h.at[pl.ds(base, KA)], st[1], sem)

        def issue_gathers(j, st, sem):
            ids_v, idd_v, xlb, xrb, eab = st[:5]
            base = e0 + j * KA
            pltpu.async_copy(xl_h.at[ids_v], xlb, sem)
            pltpu.async_copy(xr_h.at[idd_v], xrb, sem)
            pltpu.async_copy(ea_h.at[pl.ds(base, KA)], eab, sem)

        def drain_idx(st, sem):
            pltpu.make_async_copy(src_h.at[pl.ds(0, KA)], st[0], sem).wait()
            pltpu.make_async_copy(dst_h.at[pl.ds(0, KA)], st[1], sem).wait()

        def drain_gathers(st, sem):
            pltpu.make_async_copy(xl_h.at[st[0]], st[2], sem).wait()
            pltpu.make_async_copy(xr_h.at[st[1]], st[3], sem).wait()
            pltpu.make_async_copy(ea_h.at[pl.ds(0, KA)], st[4], sem).wait()

        def compute(j, st, sa):
            _, idd_v, xlb, xrb, eab, alpb = st
            base = e0 + j * KA

            def edge_body(kk, _):
                for h in range(HEADS):
                    sl = pl.ds(h * 16, 16)
                    v = xlb[kk, sl] + xrb[kk, sl] + eab[kk, sl]
                    v = jnp.maximum(v, 0.0) + 0.2 * jnp.minimum(v, 0.0)
                    tflat[pl.ds(kk * 128 + h * 16, 16)] = v * attb[sl]
                return 0
            lax.fori_loop(0, KA, edge_body, 0)

            def pair_body(p, _):
                fbase = (p * 2 + esel) * 128 + hsel * 16
                acc0 = plsc.load_gather(tflat, [fbase])
                acc1 = plsc.load_gather(tflat, [fbase + 1])
                for cc in range(2, 16, 2):
                    acc0 = acc0 + plsc.load_gather(tflat, [fbase + cc])
                    acc1 = acc1 + plsc.load_gather(tflat, [fbase + cc + 1])
                acc = acc0 + acc1
                alpb[pl.ds(p * 16, 16)] = acc
                dpair = plsc.load_gather(idd_v, [p * 2 + esel])
                midx = dpair * 8 + hsel
                old = plsc.load_gather(amaxb, [midx])
                plsc.store_scatter(amaxb, [midx], jnp.maximum(old, acc))
                return 0
            lax.fori_loop(0, KA // 2, pair_body, 0)

            pltpu.async_copy(alpb, alpha_h.at[pl.ds(base * 8, KA * 8)], sa)

        def drain_alpha(st, sa):
            pltpu.make_async_copy(st[5], alpha_h.at[pl.ds(0, KA * 8)],
                                  sa).wait()

        issue_idx(0, set0, si0)
        drain_idx(set0, si0)
        issue_gathers(0, set0, sg0)
        issue_idx(1, set1, si1)

        def body(j2, _):
            a = j2 * 2

            @pl.when(a + 1 < nch)
            def _():
                drain_idx(set1, si1)
                issue_gathers(a + 1, set1, sg1)
            drain_gathers(set0, sg0)

            @pl.when(a >= 2)
            def _():
                drain_alpha(set0, sa0)
            compute(a, set0, sa0)

            @pl.when(a + 2 < nch)
            def _():
                issue_idx(a + 2, set0, si0)
                drain_idx(set0, si0)
                issue_gathers(a + 2, set0, sg0)

            @pl.when(a + 1 < nch)
            def _():
                drain_gathers(set1, sg1)

                @pl.when(a >= 2)
                def _():
                    drain_alpha(set1, sa1)
                compute(a + 1, set1, sa1)

            @pl.when(a + 3 < nch)
            def _():
                issue_idx(a + 3, set1, si1)
            return 0
        lax.fori_loop(0, (nch + 1) // 2, body, 0)
        drain_alpha(set0, sa0)

        @pl.when(nch > 1)
        def _():
            drain_alpha(set1, sa1)
        pltpu.sync_copy(amaxb, lmax_h.at[wid])

    return k(src, dst, xl, xr, ea, att)


def _sc_accum(src, dst, xl, alpha, amax):
    e = src.shape[0]
    n = xl.shape[0]
    epw = e // NW
    nch = epw // KC
    npsc = (n // NS) // 8 * 8
    ntail = n - npsc * NS
    nfull = npsc // KC
    nrem = npsc - nfull * KC
    nd = n * 8
    dpsc = (nd // NS) // 128 * 128
    dtail = nd - dpsc * NS
    ndq = KC * 8 // 128
    mesh = plsc.VectorSubcoreMesh(core_axis_name="c", subcore_axis_name="s")

    buf = lambda: (pltpu.VMEM((KC,), jnp.int32), pltpu.VMEM((KC,), jnp.int32),
                   pltpu.VMEM((KC, 128), jnp.float32),
                   pltpu.VMEM((KC * 8,), jnp.float32),
                   pltpu.VMEM((KC, 128), jnp.float32))

    @functools.partial(
        pl.kernel,
        mesh=mesh,
        compiler_params=pltpu.CompilerParams(needs_layout_passes=False),
        out_type=(
            jax.ShapeDtypeStruct((NC, n, 128), jnp.float32),
            jax.ShapeDtypeStruct((NC, nd), jnp.float32),
        ),
        scratch_types=[
            buf(), buf(),
            pltpu.VMEM((KC * 8 + 16,), jnp.float32),
            pltpu.VMEM((ndq, 128), jnp.int32),
            pltpu.VMEM((624,), jnp.float32),
            pltpu.VMEM_SHARED((n, 128), jnp.float32),
            pltpu.VMEM_SHARED((nd,), jnp.float32),
            pltpu.SemaphoreType.DMA,
            pltpu.SemaphoreType.DMA,
            pltpu.SemaphoreType.DMA,
            pltpu.SemaphoreType.DMA,
        ],
    )
    def k(src_h, dst_h, xl_h, alpha_h, amax_h, num_h, den_h,
          set0, set1, exb, idxd, zbuf, spnum, spden,
          si0, si1, sg0, sg1):
        cid = lax.axis_index("c")
        sid = lax.axis_index("s")
        wid = sid * NC + cid
        lane = jnp.arange(16, dtype=jnp.int32)
        hsel = jnp.bitwise_and(lane, 7)
        esel = lax.shift_right_logical(lane, 3)
        e0 = wid * epw

        zrows = set0[2]
        def zb(i, _):
            for t in range(8):
                zrows[i, pl.ds(t * 16, 16)] = jnp.zeros((16,), jnp.float32)
            return 0
        lax.fori_loop(0, KC, zb, 0)
        exb[pl.ds(KC * 8, 16)] = jnp.zeros((16,), jnp.float32)
        def zx(i, _):
            zbuf[pl.ds(i * 16, 16)] = jnp.zeros((16,), jnp.float32)
            return 0
        lax.fori_loop(0, 624 // 16, zx, 0)
        r0 = sid * npsc
        for q in range(nfull):
            pltpu.sync_copy(zrows, spnum.at[pl.ds(r0 + q * KC, KC)])
        if nrem:
            pltpu.sync_copy(zrows.at[pl.ds(0, nrem)],
                            spnum.at[pl.ds(r0 + nfull * KC, nrem)])
        if ntail:
            @pl.when(sid == NS - 1)
            def _():
                pltpu.sync_copy(zrows.at[pl.ds(0, ntail)],
                                spnum.at[pl.ds(npsc * NS, ntail)])
        d0 = sid * dpsc
        for q in range(dpsc // 624):
            pltpu.sync_copy(zbuf, spden.at[pl.ds(d0 + q * 624, 624)])
        if dtail:
            @pl.when(sid == NS - 1)
            def _():
                pltpu.sync_copy(zbuf.at[pl.ds(0, dtail)],
                                spden.at[pl.ds(dpsc * NS, dtail)])
        plsc.subcore_barrier()

        def issue_idx(j, st, sem):
            base = e0 + j * KC
            pltpu.async_copy(src_h.at[pl.ds(base, KC)], st[0], sem)
            pltpu.async_copy(dst_h.at[pl.ds(base, KC)], st[1], sem)

        def issue_gathers(j, st, sem):
            ids_v, idd_v, xlb, alpb, amr = st
            base = e0 + j * KC
            pltpu.async_copy(xl_h.at[ids_v], xlb, sem)
            pltpu.async_copy(amax_h.at[idd_v], amr, sem)
            pltpu.async_copy(alpha_h.at[pl.ds(base * 8, KC * 8)], alpb, sem)

        def drain_idx(st, sem):
            pltpu.make_async_copy(src_h.at[pl.ds(0, KC)], st[0], sem).wait()
            pltpu.make_async_copy(dst_h.at[pl.ds(0, KC)], st[1], sem).wait()

        def drain_gathers(st, sem):
            pltpu.make_async_copy(xl_h.at[st[0]], st[2], sem).wait()
            pltpu.make_async_copy(amax_h.at[st[1]], st[4], sem).wait()
            pltpu.make_async_copy(alpha_h.at[pl.ds(0, KC * 8)], st[3],
                                  sem).wait()

        def compute(j, st):
            _, idd_v, xlb, alpb, amr = st

            def pair_body(p, _):
                a = alpb[pl.ds(p * 16, 16)]
                v0 = amr[p * 2, pl.ds(0, 16)]
                v1 = amr[p * 2 + 1, pl.ds(0, 16)]
                am = jnp.where(lane < 8, v0, v1)
                ex = jnp.exp(a - am)
                exb[pl.ds(p * 16, 16)] = ex
                dpair = plsc.load_gather(idd_v, [p * 2 + esel])
                midx = dpair * 8 + hsel
                q = lax.shift_right_logical(p, 3)
                off = jnp.bitwise_and(p, 7) * 16
                idxd[q, pl.ds(off, 16)] = midx
                return 0
            lax.fori_loop(0, KC // 2, pair_body, 0)

            def edge_body(kk, _):
                eoff = kk * 8
                for h in range(HEADS):
                    sl = pl.ds(h * 16, 16)
                    m = plsc.load_gather(exb, [lane * 0 + (eoff + h)])
                    xlb[kk, sl] = xlb[kk, sl] * m
                return 0
            lax.fori_loop(0, KC, edge_body, 0)

            pltpu.sync_copy(xlb, spnum.at[idd_v], add=True)
            for q in range(ndq):
                pltpu.sync_copy(exb.at[pl.ds(q * 128, 128)],
                                spden.at[idxd.at[q]], add=True)

        issue_idx(0, set0, si0)
        drain_idx(set0, si0)
        issue_gathers(0, set0, sg0)
        issue_idx(1, set1, si1)

        def body(j2, _):
            a = j2 * 2

            @pl.when(a + 1 < nch)
            def _():
                drain_idx(set1, si1)
                issue_gathers(a + 1, set1, sg1)
            drain_gathers(set0, sg0)
            compute(a, set0)

            @pl.when(a + 2 < nch)
            def _():
                issue_idx(a + 2, set0, si0)
                drain_idx(set0, si0)
                issue_gathers(a + 2, set0, sg0)

            @pl.when(a + 1 < nch)
            def _():
                drain_gathers(set1, sg1)
                compute(a + 1, set1)

            @pl.when(a + 3 < nch)
            def _():
                issue_idx(a + 3, set1, si1)
            return 0
        lax.fori_loop(0, (nch + 1) // 2, body, 0)
        plsc.subcore_barrier()

        pltpu.sync_copy(spnum.at[pl.ds(r0, npsc)],
                        num_h.at[cid, pl.ds(r0, npsc)])
        if ntail:
            @pl.when(sid == NS - 1)
            def _():
                pltpu.sync_copy(spnum.at[pl.ds(npsc * NS, ntail)],
                                num_h.at[cid, pl.ds(npsc * NS, ntail)])
        pltpu.sync_copy(spden.at[pl.ds(d0, dpsc)],
                        den_h.at[cid, pl.ds(d0, dpsc)])
        if dtail:
            @pl.when(sid == NS - 1)
            def _():
                pltpu.sync_copy(spden.at[pl.ds(dpsc * NS, dtail)],
                                den_h.at[cid, pl.ds(dpsc * NS, dtail)])

    return k(src, dst, xl, alpha, amax)


def kernel(x, edge_index, edge_sig, params):
    p = params
    r2 = lambda a: a.reshape(1, -1)
    src = edge_index[0]
    dst = edge_index[1]

    emlp = (p['W_s1'], r2(p['b_s1']), p['W_s2'], r2(p['b_s2']),
            r2(p['ln_s_g']), r2(p['ln_s_b']))
    (ea0,) = _tc_edge(edge_sig, *emlp, p['layers'][0]['W_e'])
    we12 = jnp.concatenate([lp['W_e'] for lp in p['layers'][1:]], axis=1)

    l0 = p['layers'][0]
    h, xl, xr = _tc_node0(x, p['W_in'], r2(p['b_in']), r2(l0['ln_g']),
                          r2(l0['ln_b']), l0['W_l'], r2(l0['b_l']),
                          l0['W_r'], r2(l0['b_r']))

    n = x.shape[0]
    eas = [ea0, None, None]
    for l, lp in enumerate(p['layers']):
        att = lp['att'].reshape(-1)
        alpha, lmax = _sc_logits(src, dst, xl, xr, eas[l], att)
        if l == 0:
            eas[1], eas[2] = _tc_edge(edge_sig, *emlp, we12)
        amax = _tc_amax(lmax.reshape(NW, n, 8))
        num, den = _sc_accum(src, dst, xl, alpha, amax)
        den = den.reshape(NC, n, 8)
        if l + 1 < len(p['layers']):
            np_ = p['layers'][l + 1]
            h, xl, xr = _tc_node(h, num, den, r2(lp['bias']), r2(np_['ln_g']),
                                 r2(np_['ln_b']), np_['W_l'], r2(np_['b_l']),
                                 np_['W_r'], r2(np_['b_r']))
        else:
            out = _tc_final(h, num, den, r2(lp['bias']), r2(p['ln_f_g']),
                            r2(p['ln_f_b']))
    return out


# speedup vs baseline: 1.3150x; 1.0027x over previous
import functools
import math

import jax
import jax.numpy as jnp
from jax import lax
from jax.experimental import pallas as pl
from jax.experimental.pallas import tpu as pltpu
from jax.experimental.pallas import tpu_sc as plsc

HID = 128
HEADS = 8
C = 16
NC = 2
NS = 16
NW = NC * NS
KA = 40
KC = 80
NEG = -3.0e38


def _ln(h, g, b):
    m = jnp.mean(h, axis=-1, keepdims=True)
    v = jnp.mean((h - m) * (h - m), axis=-1, keepdims=True)
    return (h - m) / jnp.sqrt(v + 1e-5) * g + b


def _tc_edge_body(es, w1, b1, w2, b2, g, b, we, o0, o1, o2):
    sf = jnp.dot(es[:], w1[:], preferred_element_type=jnp.float32) + b1[:]
    sf = 0.5 * sf * (1.0 + lax.erf(sf / math.sqrt(2.0)))
    sf = jnp.dot(sf, w2[:], preferred_element_type=jnp.float32) + b2[:]
    sf = _ln(sf, g[:], b[:])
    ea = jnp.dot(sf, we[:], preferred_element_type=jnp.float32)
    now = 0
    for o in (o0, o1, o2):
        if o is not None:
            o[:] = ea[:, now:now + 128]
            now += 128


def _tc_edge(edge_sig, w1, b1, w2, b2, g, b, we):
    e = edge_sig.shape[0]
    nl = we.shape[1] // 128
    be = 1280
    full = lambda shp: pl.BlockSpec(shp, lambda i: (0, 0))
    body = lambda es, w1_, b1_, w2_, b2_, g_, b_, we_, *outs: _tc_edge_body(
        es, w1_, b1_, w2_, b2_, g_, b_, we_,
        *(list(outs) + [None] * (3 - nl)))
    return pl.pallas_call(
        body,
        grid=(e // be,),
        in_specs=[
            pl.BlockSpec((be, 16), lambda i: (i, 0)),
            full((16, 64)), full((1, 64)), full((64, 128)), full((1, 128)),
            full((1, 128)), full((1, 128)), full((128, 128 * nl)),
        ],
        out_specs=[pl.BlockSpec((be, 128), lambda i: (i, 0))] * nl,
        out_shape=[jax.ShapeDtypeStruct((e, 128), jnp.float32)] * nl,
    )(edge_sig, w1, b1, w2, b2, g, b, we)


def _tc_node0_body(x, win, bin_, g, b, wl, bl, wr, br, ho, xlo, xro):
    h = jnp.dot(x[:], win[:], preferred_element_type=jnp.float32) + bin_[:]
    hn = _ln(h, g[:], b[:])
    ho[:] = h
    xlo[:] = jnp.dot(hn, wl[:], preferred_element_type=jnp.float32) + bl[:]
    xro[:] = jnp.dot(hn, wr[:], preferred_element_type=jnp.float32) + br[:]


def _tc_node0(x, win, bin_, g, b, wl, bl, wr, br):
    n = x.shape[0]
    bn = 1000
    full = lambda shp: pl.BlockSpec(shp, lambda i: (0, 0))
    return pl.pallas_call(
        _tc_node0_body,
        grid=(n // bn,),
        in_specs=[
            pl.BlockSpec((bn, 128), lambda i: (i, 0)),
            full((128, 128)), full((1, 128)), full((1, 128)), full((1, 128)),
            full((128, 128)), full((1, 128)), full((128, 128)), full((1, 128)),
        ],
        out_specs=[pl.BlockSpec((bn, 128), lambda i: (i, 0))] * 3,
        out_shape=[jax.ShapeDtypeStruct((n, 128), jnp.float32)] * 3,
    )(x, win, bin_, g, b, wl, bl, wr, br)


def _attn_from_parts(nump, denp, bias):
    num = nump[0] + nump[1]
    den = denp[0] + denp[1]
    cols = []
    for h in range(HEADS):
        cols.append(num[:, h * C:(h + 1) * C] / (den[:, h:h + 1] + 1e-16))
    return jnp.concatenate(cols, axis=1) + bias


def _tc_node_body(hp, nump, denp, bias, g, b, wl, bl, wr, br, ho, xlo, xro):
    h = hp[:] + _attn_from_parts(nump[:], denp[:], bias[:])
    hn = _ln(h, g[:], b[:])
    ho[:] = h
    xlo[:] = jnp.dot(hn, wl[:], preferred_element_type=jnp.float32) + bl[:]
    xro[:] = jnp.dot(hn, wr[:], preferred_element_type=jnp.float32) + br[:]


def _tc_node(hp, nump, denp, bias, g, b, wl, bl, wr, br):
    n = hp.shape[0]
    bn = 1000
    full = lambda shp: pl.BlockSpec(shp, lambda i: (0, 0))
    return pl.pallas_call(
        _tc_node_body,
        grid=(n // bn,),
        in_specs=[
            pl.BlockSpec((bn, 128), lambda i: (i, 0)),
            pl.BlockSpec((2, bn, 128), lambda i: (0, i, 0)),
            pl.BlockSpec((2, bn, 8), lambda i: (0, i, 0)),
            full((1, 128)), full((1, 128)), full((1, 128)),
            full((128, 128)), full((1, 128)), full((128, 128)), full((1, 128)),
        ],
        out_specs=[pl.BlockSpec((bn, 128), lambda i: (i, 0))] * 3,
        out_shape=[jax.ShapeDtypeStruct((n, 128), jnp.float32)] * 3,
    )(hp, nump, denp, bias, g, b, wl, bl, wr, br)


def _tc_final_body(hp, nump, denp, bias, g, b, out):
    h = hp[:] + _attn_from_parts(nump[:], denp[:], bias[:])
    out[:] = _ln(h, g[:], b[:])


def _tc_final(hp, nump, denp, bias, g, b):
    n = hp.shape[0]
    bn = 1000
    full = lambda shp: pl.BlockSpec(shp, lambda i: (0, 0))
    return pl.pallas_call(
        _tc_final_body,
        grid=(n // bn,),
        in_specs=[
            pl.BlockSpec((bn, 128), lambda i: (i, 0)),
            pl.BlockSpec((2, bn, 128), lambda i: (0, i, 0)),
            pl.BlockSpec((2, bn, 8), lambda i: (0, i, 0)),
            full((1, 128)), full((1, 128)), full((1, 128)),
        ],
        out_specs=pl.BlockSpec((bn, 128), lambda i: (i, 0)),
        out_shape=jax.ShapeDtypeStruct((n, 128), jnp.float32),
    )(hp, nump, denp, bias, g, b)


def _tc_amax_body(lmax, out):
    mx = jnp.max(lmax[:], axis=0)
    out[:] = jnp.concatenate([mx] * 16, axis=1)


def _tc_amax(lmax):
    n = lmax.shape[1]
    bn = 1000
    return pl.pallas_call(
        _tc_amax_body,
        grid=(n // bn,),
        in_specs=[pl.BlockSpec((NW, bn, 8), lambda i: (0, i, 0))],
        out_specs=pl.BlockSpec((bn, 128), lambda i: (i, 0)),
        out_shape=jax.ShapeDtypeStruct((n, 128), jnp.float32),
    )(lmax)


def _sc_logits(src, dst, xl, xr, ea, att):
    e = src.shape[0]
    n = xl.shape[0]
    epw = e // NW
    nch = epw // KA
    mesh = plsc.VectorSubcoreMesh(core_axis_name="c", subcore_axis_name="s")

    buf = lambda: (pltpu.VMEM((KA,), jnp.int32), pltpu.VMEM((KA,), jnp.int32),
                   pltpu.VMEM((KA, 128), jnp.float32),
                   pltpu.VMEM((KA, 128), jnp.float32),
                   pltpu.VMEM((KA, 128), jnp.float32),
                   pltpu.VMEM((KA * 8,), jnp.float32))

    @functools.partial(
        pl.kernel,
        mesh=mesh,
        compiler_params=pltpu.CompilerParams(needs_layout_passes=False),
        out_type=(
            jax.ShapeDtypeStruct((e * 8,), jnp.float32),
            jax.ShapeDtypeStruct((NW, n * 8), jnp.float32),
        ),
        scratch_types=[
            buf(), buf(),
            pltpu.VMEM((KA * 128,), jnp.float32),
            pltpu.VMEM((128,), jnp.float32),
            pltpu.VMEM((n * 8,), jnp.float32),
            pltpu.SemaphoreType.DMA,
            pltpu.SemaphoreType.DMA,
            pltpu.SemaphoreType.DMA,
            pltpu.SemaphoreType.DMA,
            pltpu.SemaphoreType.DMA,
            pltpu.SemaphoreType.DMA,
        ],
    )
    def k(src_h, dst_h, xl_h, xr_h, ea_h, att_h, alpha_h, lmax_h,
          set0, set1, tflat, attb, amaxb, si0, si1, sg0, sg1, sa0, sa1):
        wid = lax.axis_index("s") * NC + lax.axis_index("c")
        lane = jnp.arange(16, dtype=jnp.int32)
        hsel = jnp.bitwise_and(lane, 7)
        esel = lax.shift_right_logical(lane, 3)
        e0 = wid * epw

        pltpu.sync_copy(att_h, attb)

        def init_body(i, _):
            amaxb[pl.ds(i * 16, 16)] = jnp.full((16,), NEG, jnp.float32)
            return 0
        lax.fori_loop(0, n * 8 // 16, init_body, 0)

        def issue_idx(j, st, sem):
            base = e0 + j * KA
            pltpu.async_copy(src_h.at[pl.ds(base, KA)], st[0], sem)
            pltpu.async_copy(dst_h.at[pl.ds(base, KA)], st[1], sem)

        def issue_gathers(j, st, sem):
            ids_v, idd_v, xlb, xrb, eab = st[:5]
            base = e0 + j * KA
            pltpu.async_copy(xl_h.at[ids_v], xlb, sem)
            pltpu.async_copy(xr_h.at[idd_v], xrb, sem)
            pltpu.async_copy(ea_h.at[pl.ds(base, KA)], eab, sem)

        def drain_idx(st, sem):
            pltpu.make_async_copy(src_h.at[pl.ds(0, KA)], st[0], sem).wait()
            pltpu.make_async_copy(dst_h.at[pl.ds(0, KA)], st[1], sem).wait()

        def drain_gathers(st, sem):
            pltpu.make_async_copy(xl_h.at[st[0]], st[2], sem).wait()
            pltpu.make_async_copy(xr_h.at[st[1]], st[3], sem).wait()
            pltpu.make_async_copy(ea_h.at[pl.ds(0, KA)], st[4], sem).wait()

        def compute(j, st, sa):
            _, idd_v, xlb, xrb, eab, alpb = st
            base = e0 + j * KA

            def edge_body(pp, _):
                for sub in range(2):
                    kk = pp * 2 + sub
                    for h in range(HEADS):
                        sl = pl.ds(h * 16, 16)
                        v = xlb[kk, sl] + xrb[kk, sl] + eab[kk, sl]
                        v = jnp.maximum(v, 0.0) + 0.2 * jnp.minimum(v, 0.0)
                        tflat[pl.ds(kk * 128 + h * 16, 16)] = v * attb[sl]
                return 0
            lax.fori_loop(0, KA // 2, edge_body, 0)

            def pair_body(p, _):
                fbase = (p * 2 + esel) * 128 + hsel * 16
                acc0 = plsc.load_gather(tflat, [fbase])
                acc1 = plsc.load_gather(tflat, [fbase + 1])
                for cc in range(2, 16, 2):
                    acc0 = acc0 + plsc.load_gather(tflat, [fbase + cc])
                    acc1 = acc1 + plsc.load_gather(tflat, [fbase + cc + 1])
                acc = acc0 + acc1
                alpb[pl.ds(p * 16, 16)] = acc
                dpair = plsc.load_gather(idd_v, [p * 2 + esel])
                midx = dpair * 8 + hsel
                old = plsc.load_gather(amaxb, [midx])
                plsc.store_scatter(amaxb, [midx], jnp.maximum(old, acc))
                return 0
            lax.fori_loop(0, KA // 2, pair_body, 0)

            pltpu.async_copy(alpb, alpha_h.at[pl.ds(base * 8, KA * 8)], sa)

        def drain_alpha(st, sa):
            pltpu.make_async_copy(st[5], alpha_h.at[pl.ds(0, KA * 8)],
                                  sa).wait()

        issue_idx(0, set0, si0)
        drain_idx(set0, si0)
        issue_gathers(0, set0, sg0)
        issue_idx(1, set1, si1)

        def body(j2, _):
            a = j2 * 2

            @pl.when(a + 1 < nch)
            def _():
                drain_idx(set1, si1)
                issue_gathers(a + 1, set1, sg1)
            drain_gathers(set0, sg0)

            @pl.when(a >= 2)
            def _():
                drain_alpha(set0, sa0)
            compute(a, set0, sa0)

            @pl.when(a + 2 < nch)
            def _():
                issue_idx(a + 2, set0, si0)
                drain_idx(set0, si0)
                issue_gathers(a + 2, set0, sg0)

            @pl.when(a + 1 < nch)
            def _():
                drain_gathers(set1, sg1)

                @pl.when(a >= 2)
                def _():
                    drain_alpha(set1, sa1)
                compute(a + 1, set1, sa1)

            @pl.when(a + 3 < nch)
            def _():
                issue_idx(a + 3, set1, si1)
            return 0
        lax.fori_loop(0, (nch + 1) // 2, body, 0)
        drain_alpha(set0, sa0)

        @pl.when(nch > 1)
        def _():
            drain_alpha(set1, sa1)
        pltpu.sync_copy(amaxb, lmax_h.at[wid])

    return k(src, dst, xl, xr, ea, att)


def _sc_accum(src, dst, xl, alpha, amax):
    e = src.shape[0]
    n = xl.shape[0]
    epw = e // NW
    nch = epw // KC
    npsc = (n // NS) // 8 * 8
    ntail = n - npsc * NS
    nfull = npsc // KC
    nrem = npsc - nfull * KC
    nd = n * 8
    dpsc = (nd // NS) // 128 * 128
    dtail = nd - dpsc * NS
    ndq = KC * 8 // 128
    mesh = plsc.VectorSubcoreMesh(core_axis_name="c", subcore_axis_name="s")

    buf = lambda: (pltpu.VMEM((KC,), jnp.int32), pltpu.VMEM((KC,), jnp.int32),
                   pltpu.VMEM((KC, 128), jnp.float32),
                   pltpu.VMEM((KC * 8,), jnp.float32),
                   pltpu.VMEM((KC, 128), jnp.float32))

    @functools.partial(
        pl.kernel,
        mesh=mesh,
        compiler_params=pltpu.CompilerParams(needs_layout_passes=False),
        out_type=(
            jax.ShapeDtypeStruct((NC, n, 128), jnp.float32),
            jax.ShapeDtypeStruct((NC, nd), jnp.float32),
        ),
        scratch_types=[
            buf(), buf(),
            pltpu.VMEM((KC * 8 + 16,), jnp.float32),
            pltpu.VMEM((ndq, 128), jnp.int32),
            pltpu.VMEM((624,), jnp.float32),
            pltpu.VMEM_SHARED((n, 128), jnp.float32),
            pltpu.VMEM_SHARED((nd,), jnp.float32),
            pltpu.SemaphoreType.DMA,
            pltpu.SemaphoreType.DMA,
            pltpu.SemaphoreType.DMA,
            pltpu.SemaphoreType.DMA,
        ],
    )
    def k(src_h, dst_h, xl_h, alpha_h, amax_h, num_h, den_h,
          set0, set1, exb, idxd, zbuf, spnum, spden,
          si0, si1, sg0, sg1):
        cid = lax.axis_index("c")
        sid = lax.axis_index("s")
        wid = sid * NC + cid
        lane = jnp.arange(16, dtype=jnp.int32)
        hsel = jnp.bitwise_and(lane, 7)
        esel = lax.shift_right_logical(lane, 3)
        e0 = wid * epw

        zrows = set0[2]
        def zb(i, _):
            for t in range(8):
                zrows[i, pl.ds(t * 16, 16)] = jnp.zeros((16,), jnp.float32)
            return 0
        lax.fori_loop(0, KC, zb, 0)
        exb[pl.ds(KC * 8, 16)] = jnp.zeros((16,), jnp.float32)
        def zx(i, _):
            zbuf[pl.ds(i * 16, 16)] = jnp.zeros((16,), jnp.float32)
            return 0
        lax.fori_loop(0, 624 // 16, zx, 0)
        r0 = sid * npsc
        for q in range(nfull):
            pltpu.sync_copy(zrows, spnum.at[pl.ds(r0 + q * KC, KC)])
        if nrem:
            pltpu.sync_copy(zrows.at[pl.ds(0, nrem)],
                            spnum.at[pl.ds(r0 + nfull * KC, nrem)])
        if ntail:
            @pl.when(sid == NS - 1)
            def _():
                pltpu.sync_copy(zrows.at[pl.ds(0, ntail)],
                                spnum.at[pl.ds(npsc * NS, ntail)])
        d0 = sid * dpsc
        for q in range(dpsc // 624):
            pltpu.sync_copy(zbuf, spden.at[pl.ds(d0 + q * 624, 624)])
        if dtail:
            @pl.when(sid == NS - 1)
            def _():
                pltpu.sync_copy(zbuf.at[pl.ds(0, dtail)],
                                spden.at[pl.ds(dpsc * NS, dtail)])
        plsc.subcore_barrier()

        def issue_idx(j, st, sem):
            base = e0 + j * KC
            pltpu.async_copy(src_h.at[pl.ds(base, KC)], st[0], sem)
            pltpu.async_copy(dst_h.at[pl.ds(base, KC)], st[1], sem)

        def issue_gathers(j, st, sem):
            ids_v, idd_v, xlb, alpb, amr = st
            base = e0 + j * KC
            pltpu.async_copy(xl_h.at[ids_v], xlb, sem)
            pltpu.async_copy(amax_h.at[idd_v], amr, sem)
            pltpu.async_copy(alpha_h.at[pl.ds(base * 8, KC * 8)], alpb, sem)

        def drain_idx(st, sem):
            pltpu.make_async_copy(src_h.at[pl.ds(0, KC)], st[0], sem).wait()
            pltpu.make_async_copy(dst_h.at[pl.ds(0, KC)], st[1], sem).wait()

        def drain_gathers(st, sem):
            pltpu.make_async_copy(xl_h.at[st[0]], st[2], sem).wait()
            pltpu.make_async_copy(amax_h.at[st[1]], st[4], sem).wait()
            pltpu.make_async_copy(alpha_h.at[pl.ds(0, KC * 8)], st[3],
                                  sem).wait()

        def compute(j, st):
            _, idd_v, xlb, alpb, amr = st

            def pair_body(p, _):
                a = alpb[pl.ds(p * 16, 16)]
                v0 = amr[p * 2, pl.ds(0, 16)]
                v1 = amr[p * 2 + 1, pl.ds(0, 16)]
                am = jnp.where(lane < 8, v0, v1)
                ex = jnp.exp(a - am)
                exb[pl.ds(p * 16, 16)] = ex
                dpair = plsc.load_gather(idd_v, [p * 2 + esel])
                midx = dpair * 8 + hsel
                q = lax.shift_right_logical(p, 3)
                off = jnp.bitwise_and(p, 7) * 16
                idxd[q, pl.ds(off, 16)] = midx
                return 0
            lax.fori_loop(0, KC // 2, pair_body, 0)

            def edge_body(pp, _):
                for sub in range(2):
                    kk = pp * 2 + sub
                    eoff = kk * 8
                    for h in range(HEADS):
                        sl = pl.ds(h * 16, 16)
                        m = plsc.load_gather(exb, [lane * 0 + (eoff + h)])
                        xlb[kk, sl] = xlb[kk, sl] * m
                return 0
            lax.fori_loop(0, KC // 2, edge_body, 0)

            pltpu.sync_copy(xlb, spnum.at[idd_v], add=True)
            for q in range(ndq):
                pltpu.sync_copy(exb.at[pl.ds(q * 128, 128)],
                                spden.at[idxd.at[q]], add=True)

        issue_idx(0, set0, si0)
        drain_idx(set0, si0)
        issue_gathers(0, set0, sg0)
        issue_idx(1, set1, si1)

        def body(j2, _):
            a = j2 * 2

            @pl.when(a + 1 < nch)
            def _():
                drain_idx(set1, si1)
                issue_gathers(a + 1, set1, sg1)
            drain_gathers(set0, sg0)
            compute(a, set0)

            @pl.when(a + 2 < nch)
            def _():
                issue_idx(a + 2, set0, si0)
                drain_idx(set0, si0)
                issue_gathers(a + 2, set0, sg0)

            @pl.when(a + 1 < nch)
            def _():
                drain_gathers(set1, sg1)
                compute(a + 1, set1)

            @pl.when(a + 3 < nch)
            def _():
                issue_idx(a + 3, set1, si1)
            return 0
        lax.fori_loop(0, (nch + 1) // 2, body, 0)
        plsc.subcore_barrier()

        pltpu.sync_copy(spnum.at[pl.ds(r0, npsc)],
                        num_h.at[cid, pl.ds(r0, npsc)])
        if ntail:
            @pl.when(sid == NS - 1)
            def _():
                pltpu.sync_copy(spnum.at[pl.ds(npsc * NS, ntail)],
                                num_h.at[cid, pl.ds(npsc * NS, ntail)])
        pltpu.sync_copy(spden.at[pl.ds(d0, dpsc)],
                        den_h.at[cid, pl.ds(d0, dpsc)])
        if dtail:
            @pl.when(sid == NS - 1)
            def _():
                pltpu.sync_copy(spden.at[pl.ds(dpsc * NS, dtail)],
                                den_h.at[cid, pl.ds(dpsc * NS, dtail)])

    return k(src, dst, xl, alpha, amax)


def kernel(x, edge_index, edge_sig, params):
    p = params
    r2 = lambda a: a.reshape(1, -1)
    src = edge_index[0]
    dst = edge_index[1]

    emlp = (p['W_s1'], r2(p['b_s1']), p['W_s2'], r2(p['b_s2']),
            r2(p['ln_s_g']), r2(p['ln_s_b']))
    (ea0,) = _tc_edge(edge_sig, *emlp, p['layers'][0]['W_e'])
    we12 = jnp.concatenate([lp['W_e'] for lp in p['layers'][1:]], axis=1)

    l0 = p['layers'][0]
    h, xl, xr = _tc_node0(x, p['W_in'], r2(p['b_in']), r2(l0['ln_g']),
                          r2(l0['ln_b']), l0['W_l'], r2(l0['b_l']),
                          l0['W_r'], r2(l0['b_r']))

    n = x.shape[0]
    eas = [ea0, None, None]
    for l, lp in enumerate(p['layers']):
        att = lp['att'].reshape(-1)
        alpha, lmax = _sc_logits(src, dst, xl, xr, eas[l], att)
        if l == 0:
            eas[1], eas[2] = _tc_edge(edge_sig, *emlp, we12)
        amax = _tc_amax(lmax.reshape(NW, n, 8))
        num, den = _sc_accum(src, dst, xl, alpha, amax)
        den = den.reshape(NC, n, 8)
        if l + 1 < len(p['layers']):
            np_ = p['layers'][l + 1]
            h, xl, xr = _tc_node(h, num, den, r2(lp['bias']), r2(np_['ln_g']),
                                 r2(np_['ln_b']), np_['W_l'], r2(np_['b_l']),
                                 np_['W_r'], r2(np_['b_r']))
        else:
            out = _tc_final(h, num, den, r2(lp['bias']), r2(p['ln_f_g']),
                            r2(p['ln_f_b']))
    return out
